# Initial kernel scaffold; baseline (speedup 1.0000x reference)
import jax
import jax.numpy as jnp
from jax.experimental import pallas as pl


def kernel(x, edge_index, W1, W2, W3, b1, b2, b3, lin_W, lin_b):
    raise NotImplementedError("write your pallas kernel here")


# speedup vs baseline: 2025.1354x; 2025.1354x over previous
import jax
import jax.numpy as jnp
from jax.experimental import pallas as pl


def _gcn_kernel(adj_ref, xT_ref, w1t_ref, w2t_ref, w3t_ref,
                b1_ref, b2_ref, b3_ref, lwt_ref, lb_ref, out_ref):
    hp = jax.lax.Precision.HIGHEST

    def dot(a, b):
        return jax.lax.dot(a, b, precision=hp)

    af = adj_ref[...].astype(jnp.float32)

    h0 = dot(w1t_ref[...], xT_ref[...])
    x1 = jnp.maximum(dot(h0, af) + b1_ref[...], 0.0)
    h1 = dot(w2t_ref[...], x1)
    x2 = jnp.maximum(dot(h1, af) + b2_ref[...], 0.0)
    h2 = dot(w3t_ref[...], x2)
    x3 = dot(h2, af) + b3_ref[...]

    lw = lwt_ref[...]
    lt = (dot(lw[:, 0:20], x1) + dot(lw[:, 20:40], x2)
          + dot(lw[:, 40:60], x3) + lb_ref[...])

    m = jnp.max(lt, axis=0, keepdims=True)
    s = lt - m
    out_ref[...] = s - jnp.log(jnp.sum(jnp.exp(s), axis=0, keepdims=True))


def kernel(x, edge_index, W1, W2, W3, b1, b2, b3, lin_W, lin_b):
    n = x.shape[0]
    num_classes = lin_W.shape[1]
    outT = pl.pallas_call(
        _gcn_kernel,
        out_shape=jax.ShapeDtypeStruct((num_classes, n), jnp.float32),
    )(
        edge_index,
        x.T,
        W1.T, W2.T, W3.T,
        b1[:, None], b2[:, None], b3[:, None],
        lin_W.T,
        lin_b[:, None],
    )
    return outT.T


# speedup vs baseline: 2995.8961x; 1.4794x over previous
import jax
import jax.numpy as jnp
from jax.experimental import pallas as pl


def _gcn_kernel(adj_ref, xT_ref, w1t_ref, w2t_ref, w3t_ref,
                b1_ref, b2_ref, b3_ref, lwt_ref, lb_ref, out_ref):
    hp = jax.lax.Precision.HIGHEST

    def dot(a, b):
        return jax.lax.dot(a, b, precision=hp)

    af = adj_ref[...].astype(jnp.bfloat16)

    def agg(h):
        h_hi = h.astype(jnp.bfloat16)
        h_lo = (h - h_hi.astype(jnp.float32)).astype(jnp.bfloat16)

        def d(a):
            return jax.lax.dot_general(
                a, af, (((1,), (0,)), ((), ())),
                preferred_element_type=jnp.float32)

        return d(h_hi) + d(h_lo)

    h0 = dot(w1t_ref[...], xT_ref[...])
    x1 = jnp.maximum(agg(h0) + b1_ref[...], 0.0)
    h1 = dot(w2t_ref[...], x1)
    x2 = jnp.maximum(agg(h1) + b2_ref[...], 0.0)
    h2 = dot(w3t_ref[...], x2)
    x3 = agg(h2) + b3_ref[...]

    lw = lwt_ref[...]
    lt = (dot(lw[:, 0:20], x1) + dot(lw[:, 20:40], x2)
          + dot(lw[:, 40:60], x3) + lb_ref[...])

    m = jnp.max(lt, axis=0, keepdims=True)
    s = lt - m
    out_ref[...] = s - jnp.log(jnp.sum(jnp.exp(s), axis=0, keepdims=True))


def kernel(x, edge_index, W1, W2, W3, b1, b2, b3, lin_W, lin_b):
    n = x.shape[0]
    num_classes = lin_W.shape[1]
    outT = pl.pallas_call(
        _gcn_kernel,
        out_shape=jax.ShapeDtypeStruct((num_classes, n), jnp.float32),
    )(
        edge_index,
        x.T,
        W1.T, W2.T, W3.T,
        b1[:, None], b2[:, None], b3[:, None],
        lin_W.T,
        lin_b[:, None],
    )
    return outT.T


# speedup vs baseline: 3179.5415x; 1.0613x over previous
import jax
import jax.numpy as jnp
from jax.experimental import pallas as pl
from jax.experimental.pallas import tpu as pltpu

_N_CHUNKS = 8


def _split(h):
    h_hi = h.astype(jnp.bfloat16)
    h_lo = (h - h_hi.astype(jnp.float32)).astype(jnp.bfloat16)
    return h_hi, h_lo


def _dg(a, b):
    return jax.lax.dot_general(a, b, (((1,), (0,)), ((), ())),
                               preferred_element_type=jnp.float32)


def _gcn_kernel(adj_hbm, x_ref, w1t_ref, w2t_ref, w3t_ref,
                b1_ref, b2_ref, b3_ref, lwt_ref, lb_ref, out_ref,
                a_raw, a_bf, sems):
    n = adj_hbm.shape[0]
    chunk = n // _N_CHUNKS
    nh = w1t_ref.shape[0]

    copies = [
        pltpu.make_async_copy(
            adj_hbm.at[pl.ds(i * chunk, chunk), :],
            a_raw.at[pl.ds(i * chunk, chunk), :],
            sems.at[i])
        for i in range(_N_CHUNKS)
    ]
    for c in copies:
        c.start()

    hp = jax.lax.Precision.HIGHEST

    def dot(a, b):
        return jax.lax.dot(a, b, precision=hp)

    h0 = jax.lax.dot_general(w1t_ref[...], x_ref[...],
                             (((1,), (1,)), ((), ())), precision=hp,
                             preferred_element_type=jnp.float32)
    h0_hi, h0_lo = _split(h0)

    acc = jnp.zeros((nh, n), jnp.float32)
    for i in range(_N_CHUNKS):
        copies[i].wait()
        a_i = a_raw[pl.ds(i * chunk, chunk), :].astype(jnp.bfloat16)
        a_bf[pl.ds(i * chunk, chunk), :] = a_i
        lo, hi = i * chunk, (i + 1) * chunk
        acc = acc + _dg(h0_hi[:, lo:hi], a_i) + _dg(h0_lo[:, lo:hi], a_i)
    x1 = jnp.maximum(acc + b1_ref[...], 0.0)

    def agg(h):
        h_hi, h_lo = _split(h)
        af = a_bf[...]
        return _dg(h_hi, af) + _dg(h_lo, af)

    h1 = dot(w2t_ref[...], x1)
    x2 = jnp.maximum(agg(h1) + b2_ref[...], 0.0)
    h2 = dot(w3t_ref[...], x2)
    x3 = agg(h2) + b3_ref[...]

    lw = lwt_ref[...]
    lt = (dot(lw[:, 0:nh], x1) + dot(lw[:, nh:2 * nh], x2)
          + dot(lw[:, 2 * nh:3 * nh], x3) + lb_ref[...])

    m = jnp.max(lt, axis=0, keepdims=True)
    s = lt - m
    out_ref[...] = (s - jnp.log(jnp.sum(jnp.exp(s), axis=0, keepdims=True))).T


def kernel(x, edge_index, W1, W2, W3, b1, b2, b3, lin_W, lin_b):
    n = x.shape[0]
    num_classes = lin_W.shape[1]
    vmem = pl.BlockSpec(memory_space=pltpu.MemorySpace.VMEM)
    return pl.pallas_call(
        _gcn_kernel,
        out_shape=jax.ShapeDtypeStruct((n, num_classes), jnp.float32),
        in_specs=[pl.BlockSpec(memory_space=pl.ANY)] + [vmem] * 9,
        out_specs=vmem,
        scratch_shapes=[
            pltpu.VMEM((n, n), jnp.int32),
            pltpu.VMEM((n, n), jnp.bfloat16),
            pltpu.SemaphoreType.DMA((_N_CHUNKS,)),
        ],
    )(
        edge_index,
        x,
        W1.T, W2.T, W3.T,
        b1[:, None], b2[:, None], b3[:, None],
        lin_W.T,
        lin_b[:, None],
    )
